# baseline (speedup 1.0000x reference)
import functools

import jax
import jax.numpy as jnp
from jax import lax
from jax.experimental import pallas as pl
from jax.experimental.pallas import tpu as pltpu
from jax.experimental.pallas import tpu_sc as plsc

N_RAIN = 4 * 8 * 1 * 256 * 256
N_OBS = 4 * 8 * 200
NW = 32
ROWS = N_RAIN // 128
SHARD_ROWS = ROWS // NW
CROWS = 128
NCHUNK = SHARD_ROWS // CROWS
L1B = 32768
L2B = 65536
PCT = 95

_MESH = plsc.VectorSubcoreMesh(core_axis_name="c", subcore_axis_name="s")
_SC_PARAMS = pltpu.CompilerParams(needs_layout_passes=False)


def _zero_rows(ref, nrows):
    zf = jnp.zeros((16,), jnp.float32)

    @plsc.parallel_loop(0, nrows, unroll=4)
    def _(r):
        for j in range(8):
            ref[r, pl.ds(j * 16, 16)] = zf


def _stream_chunks(rain_hbm, row_base, bufs, sems, process):
    cp = pltpu.async_copy(
        rain_hbm.at[pl.ds(row_base, CROWS), :], bufs[0], sems[0]
    )
    for c in range(NCHUNK):
        nxt = None
        if c + 1 < NCHUNK:
            nxt = pltpu.async_copy(
                rain_hbm.at[pl.ds(row_base + (c + 1) * CROWS, CROWS), :],
                bufs[(c + 1) % 2],
                sems[(c + 1) % 2],
            )
        cp.wait()
        process(bufs[c % 2])
        cp = nxt


@functools.partial(
    pl.kernel,
    mesh=_MESH,
    compiler_params=_SC_PARAMS,
    out_type=jax.ShapeDtypeStruct((NW * (L1B // 128), 128), jnp.float32),
    scratch_types=[
        pltpu.VMEM((L1B // 128, 128), jnp.float32),
        pltpu.VMEM((CROWS, 128), jnp.float32),
        pltpu.VMEM((CROWS, 128), jnp.float32),
        pltpu.SemaphoreType.DMA,
        pltpu.SemaphoreType.DMA,
    ],
)
def _sc_hist1(rain_hbm, cnt_out, h_cnt, buf0, buf1, sem0, sem1):
    wid = lax.axis_index("s") * 2 + lax.axis_index("c")
    ones = jnp.ones((16,), jnp.float32)
    _zero_rows(h_cnt, L1B // 128)

    def process(cur):
        @plsc.parallel_loop(0, CROWS, unroll=2)
        def _(r):
            for j in range(8):
                v = cur[r, pl.ds(j * 16, 16)]
                u = plsc.bitcast(v, jnp.int32)
                m = u > 0
                row = jnp.where(m, lax.shift_right_logical(u, 23), 0)
                col = lax.shift_right_logical(u, 16) & 127
                plsc.addupdate_scatter(h_cnt, [row, col], ones, mask=m)

    _stream_chunks(rain_hbm, wid * SHARD_ROWS, (buf0, buf1), (sem0, sem1), process)
    pltpu.sync_copy(h_cnt, cnt_out.at[pl.ds(wid * (L1B // 128), L1B // 128), :])


def _tc_merge1_body(cnt_ref, ints_ref, flts_ref):
    rows = L1B // 128

    def body(t, acc):
        return acc + cnt_ref[t]

    cnt = lax.fori_loop(0, NW, body, jnp.zeros((rows, 128), jnp.float32))
    n_f = jnp.sum(cnt)
    n = n_f.astype(jnp.int32)
    k = jnp.maximum(jnp.int32(1), (n * PCT) // 100)
    k_f = k.astype(jnp.float32)

    idx = (
        lax.broadcasted_iota(jnp.int32, (rows, 128), 0) * 128
        + lax.broadcasted_iota(jnp.int32, (rows, 128), 1)
    )

    def bis(_, lo_hi):
        lo, hi = lo_hi
        mid = lo + (hi - lo) // 2
        s = jnp.sum(jnp.where(idx >= mid, cnt, 0.0))
        good = s >= k_f
        return jnp.where(good, mid, lo), jnp.where(good, hi, mid)

    lo, hi = lax.fori_loop(0, 16, bis, (jnp.int32(0), jnp.int32(L1B)))

    ints_ref[0] = n
    ints_ref[1] = k
    ints_ref[2] = lo
    for j in range(3, 16):
        ints_ref[j] = jnp.int32(0)
    flts_ref[0] = jnp.sum(jnp.where(idx > lo, cnt, 0.0))
    for j in range(1, 16):
        flts_ref[j] = jnp.float32(0)


def _tc_merge1(cnt):
    return pl.pallas_call(
        _tc_merge1_body,
        out_shape=(
            jax.ShapeDtypeStruct((16,), jnp.int32),
            jax.ShapeDtypeStruct((16,), jnp.float32),
        ),
        in_specs=[pl.BlockSpec(memory_space=pltpu.VMEM)],
        out_specs=(
            pl.BlockSpec(memory_space=pltpu.SMEM),
            pl.BlockSpec(memory_space=pltpu.SMEM),
        ),
    )(cnt)


@functools.partial(
    pl.kernel,
    mesh=_MESH,
    compiler_params=_SC_PARAMS,
    out_type=(
        jax.ShapeDtypeStruct((NW * (L2B // 128), 128), jnp.float32),
        jax.ShapeDtypeStruct((NW, 128), jnp.float32),
    ),
    scratch_types=[
        pltpu.VMEM((L2B // 128, 128), jnp.float32),
        pltpu.VMEM((CROWS, 128), jnp.float32),
        pltpu.VMEM((CROWS, 128), jnp.float32),
        pltpu.VMEM((128,), jnp.float32),
        pltpu.VMEM((16,), jnp.int32),
        pltpu.SemaphoreType.DMA,
        pltpu.SemaphoreType.DMA,
    ],
)
def _sc_hist2(rain_hbm, ints_hbm, cnt_out, stats_out, h_cnt, buf0, buf1, st, prm, sem0, sem1):
    wid = lax.axis_index("s") * 2 + lax.axis_index("c")
    pltpu.sync_copy(ints_hbm, prm)
    b15 = prm[pl.ds(0, 16)][2]
    ones = jnp.ones((16,), jnp.float32)
    zf = jnp.zeros((16,), jnp.float32)
    _zero_rows(h_cnt, L2B // 128)

    zf2 = (zf, zf)
    holder = [(zf2, zf2, zf2, zf2)]

    def process(cur):
        @plsc.parallel_loop(0, CROWS, unroll=4, carry=holder[0])
        def final(r, accs):
            accs = list(accs)
            for j in range(8):
                asum, assq = accs[j % 4]
                v = cur[r, pl.ds(j * 16, 16)]
                u = plsc.bitcast(v, jnp.int32)
                hi = lax.shift_right_arithmetic(u, 16)
                m_in = hi == b15
                m_hi = hi > b15
                row = jnp.where(m_in, lax.shift_right_arithmetic(u, 7) & 511, 0)
                col = u & 127
                plsc.addupdate_scatter(h_cnt, [row, col], ones, mask=m_in)
                w = jnp.where(m_hi, v, 0.0)
                asum = asum + w
                assq = assq + w * w
                accs[j % 4] = (asum, assq)
            return tuple(accs)

        holder[0] = final

    _stream_chunks(rain_hbm, wid * SHARD_ROWS, (buf0, buf1), (sem0, sem1), process)
    accs = holder[0]
    asum = accs[0][0] + accs[1][0] + accs[2][0] + accs[3][0]
    assq = accs[0][1] + accs[1][1] + accs[2][1] + accs[3][1]

    st[pl.ds(0, 16)] = asum
    st[pl.ds(16, 16)] = assq
    for j in range(2, 8):
        st[pl.ds(j * 16, 16)] = zf
    pltpu.sync_copy(h_cnt, cnt_out.at[pl.ds(wid * (L2B // 128), L2B // 128), :])
    pltpu.sync_copy(st, stats_out.at[wid])


def _mean_std(sum_top, ssq_top, k_f):
    mean = sum_top / k_f
    var = (ssq_top - k_f * mean * mean) / (k_f - 1.0)
    std = jnp.sqrt(jnp.maximum(var, 0.0))
    return mean, std


def _tc_merge2_body(cnt2_ref, stats_ref, ints_ref, flts_ref, obs_ref, out_ref):
    rows = L2B // 128

    def body(t, acc):
        return acc + cnt2_ref[t]

    cnt2 = lax.fori_loop(0, NW, body, jnp.zeros((rows, 128), jnp.float32))

    stats = stats_ref[...]
    col = lax.broadcasted_iota(jnp.int32, (NW, 128), 1)
    cnt_hi = flts_ref[0]
    sum_hi = jnp.sum(jnp.where(col < 16, stats, 0.0))
    ssq_hi = jnp.sum(jnp.where((col >= 16) & (col < 32), stats, 0.0))

    n = ints_ref[0]
    k = ints_ref[1]
    b15 = ints_ref[2]
    k_f = k.astype(jnp.float32)
    k2_f = k_f - cnt_hi

    idx = (
        lax.broadcasted_iota(jnp.int32, (rows, 128), 0) * 128
        + lax.broadcasted_iota(jnp.int32, (rows, 128), 1)
    )

    def bis(_, lo_hi):
        lo, hi = lo_hi
        mid = lo + (hi - lo) // 2
        s = jnp.sum(jnp.where(idx >= mid, cnt2, 0.0))
        good = s >= k2_f
        return jnp.where(good, mid, lo), jnp.where(good, hi, mid)

    lo, hi = lax.fori_loop(0, 17, bis, (jnp.int32(0), jnp.int32(L2B)))
    b2 = lo

    vals = lax.bitcast_convert_type(b15 * 65536 + idx, jnp.float32)
    above2 = idx > b2
    cnt_gt = cnt_hi + jnp.sum(jnp.where(above2, cnt2, 0.0))
    sum_gt = sum_hi + jnp.sum(jnp.where(above2, cnt2 * vals, 0.0))
    ssq_gt = ssq_hi + jnp.sum(jnp.where(above2, cnt2 * vals * vals, 0.0))
    t = jnp.sum(jnp.where(idx == b2, vals, 0.0))
    ties = k_f - cnt_gt
    gen_sum = sum_gt + ties * t
    gen_ssq = ssq_gt + ties * t * t
    gen_mean, gen_std = _mean_std(gen_sum, gen_ssq, k_f)

    obs = obs_ref[...]
    u_o = lax.bitcast_convert_type(obs, jnp.int32)
    valid = obs > 0.0
    n_o_f = jnp.sum(jnp.where(valid, 1.0, 0.0))
    n_o = n_o_f.astype(jnp.int32)
    k_o = jnp.maximum(jnp.int32(1), (n_o * PCT) // 100)
    k_o_f = k_o.astype(jnp.float32)

    def bis_o(_, lo_hi):
        lo, hi = lo_hi
        mid = lo + (hi - lo) // 2
        c = jnp.sum(jnp.where(valid & (u_o >= mid), 1.0, 0.0))
        good = c >= k_o_f
        return jnp.where(good, mid, lo), jnp.where(good, hi, mid)

    lo_o, hi_o = lax.fori_loop(
        0, 31, bis_o, (jnp.int32(1), jnp.int32(0x7F800001))
    )
    gt_o = valid & (u_o > lo_o)
    eq_o = valid & (u_o == lo_o)
    cnt_gt_o = jnp.sum(jnp.where(gt_o, 1.0, 0.0))
    sum_gt_o = jnp.sum(jnp.where(gt_o, obs, 0.0))
    ssq_gt_o = jnp.sum(jnp.where(gt_o, obs * obs, 0.0))
    t_o = jnp.max(jnp.where(eq_o, obs, -jnp.inf))
    ties_o = k_o_f - cnt_gt_o
    obs_sum = sum_gt_o + ties_o * t_o
    obs_ssq = ssq_gt_o + ties_o * t_o * t_o
    obs_mean, obs_std = _mean_std(obs_sum, obs_ssq, k_o_f)

    mean_loss = (gen_mean - obs_mean) ** 2
    std_loss = (gen_std - obs_std) ** 2
    total = mean_loss + 0.5 * std_loss
    out_ref[0] = jnp.where((n > 0) & (n_o > 0), total, jnp.float32(0.0))


def _tc_merge2(cnt2, stats, ints, flts, obs):
    return pl.pallas_call(
        _tc_merge2_body,
        out_shape=jax.ShapeDtypeStruct((1,), jnp.float32),
        in_specs=[
            pl.BlockSpec(memory_space=pltpu.VMEM),
            pl.BlockSpec(memory_space=pltpu.VMEM),
            pl.BlockSpec(memory_space=pltpu.SMEM),
            pl.BlockSpec(memory_space=pltpu.SMEM),
            pl.BlockSpec(memory_space=pltpu.VMEM),
        ],
        out_specs=pl.BlockSpec(memory_space=pltpu.SMEM),
    )(cnt2, stats, ints, flts, obs)


def kernel(rain_hr, s_values):
    rain = lax.optimization_barrier(jnp.reshape(rain_hr, (ROWS, 128)))
    obs = jnp.reshape(s_values, (N_OBS // 128, 128))
    h_cnt = _sc_hist1(rain)
    ints, flts = _tc_merge1(jnp.reshape(h_cnt, (NW, L1B // 128, 128)))
    cnt2, stats = _sc_hist2(rain, ints)
    out = _tc_merge2(
        jnp.reshape(cnt2, (NW, L2B // 128, 128)), stats, ints, flts, obs
    )
    return jnp.reshape(out, ())


# speedup vs baseline: 1.0282x; 1.0282x over previous
import functools

import jax
import jax.numpy as jnp
from jax import lax
from jax.experimental import pallas as pl
from jax.experimental.pallas import tpu as pltpu
from jax.experimental.pallas import tpu_sc as plsc

N_RAIN = 4 * 8 * 1 * 256 * 256
N_OBS = 4 * 8 * 200
NW = 32
ROWS = N_RAIN // 128
SHARD_ROWS = ROWS // NW
CROWS = 128
NCHUNK = SHARD_ROWS // CROWS
L1B = 32768
L2B = 65536
PCT = 95

_MESH = plsc.VectorSubcoreMesh(core_axis_name="c", subcore_axis_name="s")
_SC_PARAMS = pltpu.CompilerParams(needs_layout_passes=False)


def _zero_rows(ref, nrows):
    zf = jnp.zeros((16,), jnp.float32)

    @plsc.parallel_loop(0, nrows, unroll=4)
    def _(r):
        for j in range(8):
            ref[r, pl.ds(j * 16, 16)] = zf


def _stream_chunks(rain_hbm, row_base, bufs, sems, process):
    cp = pltpu.async_copy(
        rain_hbm.at[pl.ds(row_base, CROWS), :], bufs[0], sems[0]
    )
    for c in range(NCHUNK):
        nxt = None
        if c + 1 < NCHUNK:
            nxt = pltpu.async_copy(
                rain_hbm.at[pl.ds(row_base + (c + 1) * CROWS, CROWS), :],
                bufs[(c + 1) % 2],
                sems[(c + 1) % 2],
            )
        cp.wait()
        process(bufs[c % 2])
        cp = nxt


@functools.partial(
    pl.kernel,
    mesh=_MESH,
    compiler_params=_SC_PARAMS,
    out_type=jax.ShapeDtypeStruct((NW * (L1B // 128), 128), jnp.float32),
    scratch_types=[
        pltpu.VMEM((L1B // 128, 128), jnp.float32),
        pltpu.VMEM((CROWS, 128), jnp.float32),
        pltpu.VMEM((CROWS, 128), jnp.float32),
        pltpu.SemaphoreType.DMA,
        pltpu.SemaphoreType.DMA,
    ],
)
def _sc_hist1(rain_hbm, cnt_out, h_cnt, buf0, buf1, sem0, sem1):
    wid = lax.axis_index("s") * 2 + lax.axis_index("c")
    ones = jnp.ones((16,), jnp.float32)
    _zero_rows(h_cnt, L1B // 128)

    def process(cur):
        @plsc.parallel_loop(0, CROWS, unroll=2)
        def _(r):
            for j in range(8):
                v = cur[r, pl.ds(j * 16, 16)]
                u = plsc.bitcast(v, jnp.int32)
                m = u > 0
                row = jnp.where(m, lax.shift_right_logical(u, 23), 0)
                col = lax.shift_right_logical(u, 16) & 127
                plsc.addupdate_scatter(h_cnt, [row, col], ones, mask=m)

    _stream_chunks(rain_hbm, wid * SHARD_ROWS, (buf0, buf1), (sem0, sem1), process)
    pltpu.sync_copy(h_cnt, cnt_out.at[pl.ds(wid * (L1B // 128), L1B // 128), :])


def _tc_merge1_body(cnt_ref, ints_ref, flts_ref):
    rows = L1B // 128

    def body(t, acc):
        return acc + cnt_ref[t]

    cnt = lax.fori_loop(0, NW, body, jnp.zeros((rows, 128), jnp.float32))
    n_f = jnp.sum(cnt)
    n = n_f.astype(jnp.int32)
    k = jnp.maximum(jnp.int32(1), (n * PCT) // 100)
    k_f = k.astype(jnp.float32)

    idx = (
        lax.broadcasted_iota(jnp.int32, (rows, 128), 0) * 128
        + lax.broadcasted_iota(jnp.int32, (rows, 128), 1)
    )

    def bis(_, lo_hi):
        lo, hi = lo_hi
        mid = lo + (hi - lo) // 2
        s = jnp.sum(jnp.where(idx >= mid, cnt, 0.0))
        good = s >= k_f
        return jnp.where(good, mid, lo), jnp.where(good, hi, mid)

    lo, hi = lax.fori_loop(0, 16, bis, (jnp.int32(0), jnp.int32(L1B)))

    ints_ref[0] = n
    ints_ref[1] = k
    ints_ref[2] = lo
    for j in range(3, 16):
        ints_ref[j] = jnp.int32(0)
    flts_ref[0] = jnp.sum(jnp.where(idx > lo, cnt, 0.0))
    for j in range(1, 16):
        flts_ref[j] = jnp.float32(0)


def _tc_merge1(cnt):
    return pl.pallas_call(
        _tc_merge1_body,
        out_shape=(
            jax.ShapeDtypeStruct((16,), jnp.int32),
            jax.ShapeDtypeStruct((16,), jnp.float32),
        ),
        in_specs=[pl.BlockSpec(memory_space=pltpu.VMEM)],
        out_specs=(
            pl.BlockSpec(memory_space=pltpu.SMEM),
            pl.BlockSpec(memory_space=pltpu.SMEM),
        ),
    )(cnt)


@functools.partial(
    pl.kernel,
    mesh=_MESH,
    compiler_params=_SC_PARAMS,
    out_type=(
        jax.ShapeDtypeStruct((NW * (L2B // 128), 128), jnp.float32),
        jax.ShapeDtypeStruct((NW, 128), jnp.float32),
    ),
    scratch_types=[
        pltpu.VMEM((L2B // 128, 128), jnp.float32),
        pltpu.VMEM((CROWS, 128), jnp.float32),
        pltpu.VMEM((CROWS, 128), jnp.float32),
        pltpu.VMEM((128,), jnp.float32),
        pltpu.VMEM((16,), jnp.int32),
        pltpu.SemaphoreType.DMA,
        pltpu.SemaphoreType.DMA,
    ],
)
def _sc_hist2(rain_hbm, ints_hbm, cnt_out, stats_out, h_cnt, buf0, buf1, st, prm, sem0, sem1):
    wid = lax.axis_index("s") * 2 + lax.axis_index("c")
    pltpu.sync_copy(ints_hbm, prm)
    b15 = prm[pl.ds(0, 16)][2]
    ones = jnp.ones((16,), jnp.float32)
    zf = jnp.zeros((16,), jnp.float32)
    _zero_rows(h_cnt, L2B // 128)

    zf2 = (zf, zf)
    holder = [(zf2, zf2, zf2, zf2)]

    def process(cur):
        @plsc.parallel_loop(0, CROWS, unroll=2, carry=holder[0])
        def final(r, accs):
            accs = list(accs)
            for j in range(8):
                asum, assq = accs[j % 4]
                v = cur[r, pl.ds(j * 16, 16)]
                u = plsc.bitcast(v, jnp.int32)
                hi = lax.shift_right_arithmetic(u, 16)
                m_in = hi == b15
                m_hi = hi > b15
                row = jnp.where(m_in, lax.shift_right_arithmetic(u, 7) & 511, 0)
                col = u & 127
                plsc.addupdate_scatter(h_cnt, [row, col], ones, mask=m_in)
                w = jnp.where(m_hi, v, 0.0)
                asum = asum + w
                assq = assq + w * w
                accs[j % 4] = (asum, assq)
            return tuple(accs)

        holder[0] = final

    _stream_chunks(rain_hbm, wid * SHARD_ROWS, (buf0, buf1), (sem0, sem1), process)
    accs = holder[0]
    asum = accs[0][0] + accs[1][0] + accs[2][0] + accs[3][0]
    assq = accs[0][1] + accs[1][1] + accs[2][1] + accs[3][1]

    st[pl.ds(0, 16)] = asum
    st[pl.ds(16, 16)] = assq
    for j in range(2, 8):
        st[pl.ds(j * 16, 16)] = zf
    pltpu.sync_copy(h_cnt, cnt_out.at[pl.ds(wid * (L2B // 128), L2B // 128), :])
    pltpu.sync_copy(st, stats_out.at[wid])


def _mean_std(sum_top, ssq_top, k_f):
    mean = sum_top / k_f
    var = (ssq_top - k_f * mean * mean) / (k_f - 1.0)
    std = jnp.sqrt(jnp.maximum(var, 0.0))
    return mean, std


def _tc_merge2_body(cnt2_ref, stats_ref, ints_ref, flts_ref, obs_ref, out_ref):
    rows = L2B // 128

    def body(t, acc):
        return acc + cnt2_ref[t]

    cnt2 = lax.fori_loop(0, NW, body, jnp.zeros((rows, 128), jnp.float32))

    stats = stats_ref[...]
    col = lax.broadcasted_iota(jnp.int32, (NW, 128), 1)
    cnt_hi = flts_ref[0]
    sum_hi = jnp.sum(jnp.where(col < 16, stats, 0.0))
    ssq_hi = jnp.sum(jnp.where((col >= 16) & (col < 32), stats, 0.0))

    n = ints_ref[0]
    k = ints_ref[1]
    b15 = ints_ref[2]
    k_f = k.astype(jnp.float32)
    k2_f = k_f - cnt_hi

    idx = (
        lax.broadcasted_iota(jnp.int32, (rows, 128), 0) * 128
        + lax.broadcasted_iota(jnp.int32, (rows, 128), 1)
    )

    def bis(_, lo_hi):
        lo, hi = lo_hi
        mid = lo + (hi - lo) // 2
        s = jnp.sum(jnp.where(idx >= mid, cnt2, 0.0))
        good = s >= k2_f
        return jnp.where(good, mid, lo), jnp.where(good, hi, mid)

    lo, hi = lax.fori_loop(0, 17, bis, (jnp.int32(0), jnp.int32(L2B)))
    b2 = lo

    vals = lax.bitcast_convert_type(b15 * 65536 + idx, jnp.float32)
    above2 = idx > b2
    cnt_gt = cnt_hi + jnp.sum(jnp.where(above2, cnt2, 0.0))
    sum_gt = sum_hi + jnp.sum(jnp.where(above2, cnt2 * vals, 0.0))
    ssq_gt = ssq_hi + jnp.sum(jnp.where(above2, cnt2 * vals * vals, 0.0))
    t = jnp.sum(jnp.where(idx == b2, vals, 0.0))
    ties = k_f - cnt_gt
    gen_sum = sum_gt + ties * t
    gen_ssq = ssq_gt + ties * t * t
    gen_mean, gen_std = _mean_std(gen_sum, gen_ssq, k_f)

    obs = obs_ref[...]
    u_o = lax.bitcast_convert_type(obs, jnp.int32)
    valid = obs > 0.0
    n_o_f = jnp.sum(jnp.where(valid, 1.0, 0.0))
    n_o = n_o_f.astype(jnp.int32)
    k_o = jnp.maximum(jnp.int32(1), (n_o * PCT) // 100)
    k_o_f = k_o.astype(jnp.float32)

    def bis_o(_, lo_hi):
        lo, hi = lo_hi
        mid = lo + (hi - lo) // 2
        c = jnp.sum(jnp.where(valid & (u_o >= mid), 1.0, 0.0))
        good = c >= k_o_f
        return jnp.where(good, mid, lo), jnp.where(good, hi, mid)

    lo_o, hi_o = lax.fori_loop(
        0, 31, bis_o, (jnp.int32(1), jnp.int32(0x7F800001))
    )
    gt_o = valid & (u_o > lo_o)
    eq_o = valid & (u_o == lo_o)
    cnt_gt_o = jnp.sum(jnp.where(gt_o, 1.0, 0.0))
    sum_gt_o = jnp.sum(jnp.where(gt_o, obs, 0.0))
    ssq_gt_o = jnp.sum(jnp.where(gt_o, obs * obs, 0.0))
    t_o = jnp.max(jnp.where(eq_o, obs, -jnp.inf))
    ties_o = k_o_f - cnt_gt_o
    obs_sum = sum_gt_o + ties_o * t_o
    obs_ssq = ssq_gt_o + ties_o * t_o * t_o
    obs_mean, obs_std = _mean_std(obs_sum, obs_ssq, k_o_f)

    mean_loss = (gen_mean - obs_mean) ** 2
    std_loss = (gen_std - obs_std) ** 2
    total = mean_loss + 0.5 * std_loss
    out_ref[0] = jnp.where((n > 0) & (n_o > 0), total, jnp.float32(0.0))


def _tc_merge2(cnt2, stats, ints, flts, obs):
    return pl.pallas_call(
        _tc_merge2_body,
        out_shape=jax.ShapeDtypeStruct((1,), jnp.float32),
        in_specs=[
            pl.BlockSpec(memory_space=pltpu.VMEM),
            pl.BlockSpec(memory_space=pltpu.VMEM),
            pl.BlockSpec(memory_space=pltpu.SMEM),
            pl.BlockSpec(memory_space=pltpu.SMEM),
            pl.BlockSpec(memory_space=pltpu.VMEM),
        ],
        out_specs=pl.BlockSpec(memory_space=pltpu.SMEM),
    )(cnt2, stats, ints, flts, obs)


def kernel(rain_hr, s_values):
    rain = lax.optimization_barrier(jnp.reshape(rain_hr, (ROWS, 128)))
    obs = jnp.reshape(s_values, (N_OBS // 128, 128))
    h_cnt = _sc_hist1(rain)
    ints, flts = _tc_merge1(jnp.reshape(h_cnt, (NW, L1B // 128, 128)))
    cnt2, stats = _sc_hist2(rain, ints)
    out = _tc_merge2(
        jnp.reshape(cnt2, (NW, L2B // 128, 128)), stats, ints, flts, obs
    )
    return jnp.reshape(out, ())
